# baseline (speedup 1.0000x reference)
import jax
import jax.numpy as jnp
from jax.experimental import pallas as pl
from jax.experimental.pallas import tpu as pltpu

_EPS = float(jnp.finfo(jnp.float32).eps)
_HI = jax.lax.Precision.HIGHEST


def _matmul_kernel(x_ref, w_ref, out_ref):
    out_ref[...] = jax.lax.dot_general(
        w_ref[...], x_ref[...],
        dimension_numbers=(((1,), (1,)), ((), ())),
        preferred_element_type=jnp.float32,
    )


def _colsum(p, ones_e):
    return jax.lax.dot_general(
        p, ones_e, dimension_numbers=(((0,), (0,)), ((), ())),
        preferred_element_type=jnp.float32, precision=_HI)


def _gating_kernel(lt_ref, laux_ref, combine_ref, dispatch_ref):
    E, S = lt_ref.shape
    C = combine_ref.shape[1]
    lt = lt_ref[...]

    cmax = jnp.max(lt, axis=0, keepdims=True)
    unnorm = jnp.exp(lt - cmax)
    gates = unnorm / jnp.sum(unnorm, axis=0, keepdims=True)

    eidx = jax.lax.broadcasted_iota(jnp.int32, (E, S), 0)
    gmax = jnp.max(gates, axis=0, keepdims=True)
    idx1 = jnp.min(jnp.where(gates == gmax, eidx, E), axis=0, keepdims=True)
    mask1 = eidx == idx1
    masked = jnp.where(mask1, -jnp.inf, lt)
    mmax = jnp.max(masked, axis=0, keepdims=True)
    idx2 = jnp.min(jnp.where(masked == mmax, eidx, E), axis=0, keepdims=True)
    mask2 = eidx == idx2
    m1f = mask1.astype(jnp.float32)
    m2f = mask2.astype(jnp.float32)

    B = 256
    ri = jax.lax.broadcasted_iota(jnp.int32, (B, B), 0)
    ci = jax.lax.broadcasted_iota(jnp.int32, (B, B), 1)
    tri = (ri <= ci).astype(jnp.float32)
    ones_e = jnp.ones((E, 1), jnp.float32)

    def blocked(m):
        run = jnp.zeros((E, 1), jnp.float32)
        parts = []
        for b in range(S // B):
            blk = m[:, b * B:(b + 1) * B]
            cs = jax.lax.dot_general(
                blk, tri, dimension_numbers=(((1,), (0,)), ((), ())),
                preferred_element_type=jnp.float32) + run
            parts.append(_colsum((cs - 1.0) * blk, ones_e))
            run = run + jnp.sum(blk, axis=1, keepdims=True)
        return jnp.concatenate(parts, axis=0), run

    loc1, tot1 = blocked(m1f)
    loc2, _ = blocked(m2f)
    tot1_tok = jax.lax.dot_general(
        m2f, tot1, dimension_numbers=(((0,), (0,)), ((), ())),
        preferred_element_type=jnp.float32, precision=_HI)
    loc2 = loc2 + tot1_tok

    g1 = _colsum(gates * m1f, ones_e)
    g2 = _colsum(gates * m2f, ones_e)

    me = jnp.sum(gates, axis=1, keepdims=True) / S
    ce = tot1 / S
    laux_ref[...] = jnp.sum(me * ce, axis=0, keepdims=True) / E

    keep1 = (loc1 < C).astype(jnp.float32)
    keep2 = (loc2 < C).astype(jnp.float32)
    g1k = g1 * keep1
    g2k = g2 * keep2
    denom = jnp.maximum(g1k + g2k, jnp.float32(_EPS))
    g1n = g1k / denom
    g2n = g2k / denom
    l1 = (loc1 * keep1).astype(jnp.int32)
    l2 = (loc2 * keep2).astype(jnp.int32)
    cap = jax.lax.broadcasted_iota(jnp.int32, (S, C), 1)
    combine = (g1n * (cap == l1).astype(jnp.float32)
               + g2n * (cap == l2).astype(jnp.float32))
    combine_ref[...] = combine
    dispatch_ref[...] = combine != 0.0


def kernel(input, W):
    S, D = input.shape
    E = W.shape[0]
    C = 2 * S // E
    RB = 512

    logits_t = pl.pallas_call(
        _matmul_kernel,
        grid=(S // RB,),
        in_specs=[
            pl.BlockSpec((RB, D), lambda i: (i, 0)),
            pl.BlockSpec((E, D), lambda i: (0, 0)),
        ],
        out_specs=pl.BlockSpec((E, RB), lambda i: (0, i)),
        out_shape=jax.ShapeDtypeStruct((E, S), jnp.float32),
    )(input, W)

    laux, combine, dispatch = pl.pallas_call(
        _gating_kernel,
        out_shape=[
            jax.ShapeDtypeStruct((1, 1), jnp.float32),
            jax.ShapeDtypeStruct((S, C), jnp.float32),
            jax.ShapeDtypeStruct((S, C), jnp.bool_),
        ],
    )(logits_t)

    return laux[0, 0], combine.reshape(S, 1, C), dispatch.reshape(S, 1, C)


# speedup vs baseline: 1.0711x; 1.0711x over previous
import jax
import jax.numpy as jnp
from jax.experimental import pallas as pl
from jax.experimental.pallas import tpu as pltpu


def _matmul_kernel(x_ref, w_ref, out_ref):
    out_ref[...] = jax.lax.dot_general(
        x_ref[...], w_ref[...],
        dimension_numbers=(((1,), (1,)), ((), ())),
        preferred_element_type=jnp.float32,
    )


def _gating_kernel(logits_ref, laux_ref, combine_ref, dispatch_ref):
    S, E = logits_ref.shape
    C = combine_ref.shape[1]
    logits = logits_ref[...]

    row_max = jnp.max(logits, axis=1, keepdims=True)
    unnorm = jnp.exp(logits - row_max)
    gates = unnorm / jnp.sum(unnorm, axis=1, keepdims=True)

    eidx = jax.lax.broadcasted_iota(jnp.int32, (S, E), 1)
    gmax = jnp.max(gates, axis=1, keepdims=True)
    idx1 = jnp.min(jnp.where(gates == gmax, eidx, E), axis=1, keepdims=True)
    mask1 = eidx == idx1
    masked = jnp.where(mask1, -jnp.inf, logits)
    mmax = jnp.max(masked, axis=1, keepdims=True)
    idx2 = jnp.min(jnp.where(masked == mmax, eidx, E), axis=1, keepdims=True)
    mask2 = eidx == idx2
    m1f = mask1.astype(jnp.float32)
    m2f = mask2.astype(jnp.float32)

    B = 256
    ri = jax.lax.broadcasted_iota(jnp.int32, (B, B), 0)
    ci = jax.lax.broadcasted_iota(jnp.int32, (B, B), 1)
    tri = (ri >= ci).astype(jnp.float32)

    def blocked_cumsum(m):
        parts = []
        run = jnp.zeros((1, E), jnp.float32)
        for b in range(S // B):
            blk = m[b * B:(b + 1) * B]
            cs = jax.lax.dot_general(
                tri, blk, dimension_numbers=(((1,), (0,)), ((), ())),
                preferred_element_type=jnp.float32) + run
            parts.append(cs)
            run = run + jnp.sum(blk, axis=0, keepdims=True)
        return jnp.concatenate(parts, axis=0), run

    c1, tot1 = blocked_cumsum(m1f)
    c2, _ = blocked_cumsum(m2f)
    loc1 = c1 - 1.0
    loc2 = c2 - 1.0 + tot1

    me = jnp.sum(gates, axis=0, keepdims=True) / S
    ce = jnp.sum(m1f, axis=0, keepdims=True) / S
    laux_ref[...] = jnp.sum(me * ce, axis=1, keepdims=True) / E

    keep1 = (mask1 & (loc1 < C)).astype(jnp.float32)
    keep2 = (mask2 & (loc2 < C)).astype(jnp.float32)
    g1 = jnp.sum(gates * keep1, axis=1, keepdims=True)
    g2 = jnp.sum(gates * keep2, axis=1, keepdims=True)
    denom = jnp.maximum(g1 + g2, jnp.float32(jnp.finfo(jnp.float32).eps))
    g1n = g1 / denom
    g2n = g2 / denom
    l1 = jnp.sum(loc1 * keep1, axis=1, keepdims=True).astype(jnp.int32)
    l2 = jnp.sum(loc2 * keep2, axis=1, keepdims=True).astype(jnp.int32)
    cap = jax.lax.broadcasted_iota(jnp.int32, (S, C), 1)
    combine = (g1n * (cap == l1).astype(jnp.float32)
               + g2n * (cap == l2).astype(jnp.float32))
    combine_ref[...] = combine
    dispatch_ref[...] = combine != 0.0


def kernel(input, W):
    S, D = input.shape
    E = W.shape[0]
    C = 2 * S // E
    RB = 512

    logits = pl.pallas_call(
        _matmul_kernel,
        grid=(S // RB,),
        in_specs=[
            pl.BlockSpec((RB, D), lambda i: (i, 0)),
            pl.BlockSpec((E, D), lambda i: (0, 0)),
        ],
        out_specs=pl.BlockSpec((RB, E), lambda i: (i, 0)),
        out_shape=jax.ShapeDtypeStruct((S, E), jnp.float32),
    )(input, W)

    laux, combine, dispatch = pl.pallas_call(
        _gating_kernel,
        out_shape=[
            jax.ShapeDtypeStruct((1, 1), jnp.float32),
            jax.ShapeDtypeStruct((S, C), jnp.float32),
            jax.ShapeDtypeStruct((S, C), jnp.bool_),
        ],
    )(logits)

    return laux[0, 0], combine.reshape(S, 1, C), dispatch.reshape(S, 1, C)
